# baseline (device time: 26401 ns/iter reference)
import os

import jax
import jax.numpy as jnp
from jax import lax
from jax.experimental import pallas as pl
from jax.experimental.pallas import tpu as pltpu

N_DEV = 4
B = 2
SQ_LOC = 128
D_MODEL = 512
HQ = 16
HQ_LOC = 4
DH = 64
SKV = 128
BLK = 64
GDIM = HQ_LOC * DH


def kernel(x, Wq, K_ext, V_ext, Wo):
    bf16 = jnp.bfloat16
    f32 = jnp.float32

    x2 = (x.reshape(B * SQ_LOC, D_MODEL) * 0.125).astype(bf16)
    kt = jnp.transpose(K_ext, (2, 0, 1, 3)).astype(bf16)
    vt = jnp.transpose(V_ext, (2, 0, 1, 3)).astype(bf16)
    wq = Wq.astype(bf16)
    wo = Wo.astype(bf16)

    def body(x_ref, wq_ref, kt_ref, vt_ref, wo_ref, out_ref,
             wq_slots, wo_slots, k_bd, v_bd,
             wq_send, wq_recv, wo_send, wo_recv):
        my = lax.axis_index("i")
        left = lax.rem(my + N_DEV - 1, N_DEV)
        right = lax.rem(my + 1, N_DEV)
        opp = lax.rem(my + 2, N_DEV)

        k_bd[...] = jnp.zeros((HQ_LOC * SKV, GDIM), bf16)
        v_bd[...] = jnp.zeros((HQ_LOC * SKV, GDIM), bf16)

        barrier_sem = pltpu.get_barrier_semaphore()
        for nbr in (left, right, opp):
            pl.semaphore_signal(
                barrier_sem, inc=1,
                device_id=(nbr,), device_id_type=pl.DeviceIdType.MESH,
            )
        pl.semaphore_wait(barrier_sem, N_DEV - 1)

        qi = lax.broadcasted_iota(jnp.int32, (SQ_LOC, HQ_LOC * SKV), 0)
        kj = lax.broadcasted_iota(jnp.int32, (SQ_LOC, HQ_LOC * SKV), 1)
        qb = my * (SQ_LOC // BLK) + qi // BLK
        kb = lax.rem(kj, SKV) // BLK
        mask = (qb == kb) | (kb == 0) | (lax.rem(qb + kb, 3) == 0)

        def compute_group(g, wq_g, wo_g, acc):
            q_g = jax.lax.dot_general(
                x_ref[...], wq_g, (((1,), (0,)), ((), ())),
                preferred_element_type=f32,
            ).astype(bf16)
            ctxs = []
            for b in range(B):
                for hh in range(HQ_LOC):
                    head = g * HQ_LOC + hh
                    kv_h = kt_ref[head]
                    vv_h = vt_ref[head]
                    k_bd[hh * SKV:(hh + 1) * SKV, hh * DH:(hh + 1) * DH] = kv_h[b]
                    v_bd[hh * SKV:(hh + 1) * SKV, hh * DH:(hh + 1) * DH] = vv_h[b]
                q_b = q_g[b * SQ_LOC:(b + 1) * SQ_LOC, :]
                s = jax.lax.dot_general(
                    q_b, k_bd[...], (((1,), (1,)), ((), ())),
                    preferred_element_type=f32,
                )
                s = jnp.where(mask, s, -1e9)
                s3 = s.reshape(SQ_LOC, HQ_LOC, SKV)
                m = jnp.max(s3, axis=-1, keepdims=True)
                w = jnp.exp(s3 - m)
                w = w / jnp.sum(w, axis=-1, keepdims=True)
                w2 = w.reshape(SQ_LOC, HQ_LOC * SKV).astype(bf16)
                ctxs.append(jax.lax.dot_general(
                    w2, v_bd[...], (((1,), (0,)), ((), ())),
                    preferred_element_type=f32,
                ).astype(bf16))
            ctx = jnp.concatenate(ctxs, axis=0)
            return acc + jax.lax.dot_general(
                ctx, wo_g, (((1,), (0,)), ((), ())),
                preferred_element_type=f32,
            )

        acc = jnp.zeros((B * SQ_LOC, D_MODEL), dtype=f32)
        skip_comm = bool(os.environ.get("SKIP_COMM"))

        txs = []

        def push_to(dest):
            for src, slots, ssem, rsem in (
                (wq_ref, wq_slots, wq_send, wq_recv),
                (wo_ref, wo_slots, wo_send, wo_recv),
            ):
                tx = pltpu.make_async_remote_copy(
                    src_ref=src, dst_ref=slots.at[my],
                    send_sem=ssem.at[dest], recv_sem=rsem.at[my],
                    device_id=(dest,), device_id_type=pl.DeviceIdType.MESH,
                )
                tx.start()
                txs.append(tx)

        def wait_from(origin):
            for slots, rsem in ((wq_slots, wq_recv), (wo_slots, wo_recv)):
                rx = pltpu.make_async_remote_copy(
                    src_ref=slots.at[origin], dst_ref=slots.at[origin],
                    send_sem=wq_send.at[origin], recv_sem=rsem.at[origin],
                    device_id=(origin,), device_id_type=pl.DeviceIdType.MESH,
                )
                rx.wait_recv()

        if skip_comm:
            for g in range(N_DEV):
                acc = compute_group(g, wq_ref[...], wo_ref[...], acc)
        else:
            push_to(opp)
            push_to(right)
            push_to(left)
            acc = compute_group(my, wq_ref[...], wo_ref[...], acc)
            for origin in (left, right, opp):
                wait_from(origin)
                acc = compute_group(origin, wq_slots[origin],
                                    wo_slots[origin], acc)
            for tx in txs:
                tx.wait_send()

        out_ref[...] = acc

    out = pl.pallas_call(
        body,
        out_shape=jax.ShapeDtypeStruct((B * SQ_LOC, D_MODEL), jnp.float32),
        in_specs=[pl.BlockSpec(memory_space=pltpu.VMEM)] * 5,
        out_specs=pl.BlockSpec(memory_space=pltpu.VMEM),
        scratch_shapes=[
            pltpu.VMEM((N_DEV, D_MODEL, GDIM), bf16),
            pltpu.VMEM((N_DEV, GDIM, D_MODEL), bf16),
            pltpu.VMEM((HQ_LOC * SKV, GDIM), bf16),
            pltpu.VMEM((HQ_LOC * SKV, GDIM), bf16),
            pltpu.SemaphoreType.DMA((N_DEV,)),
            pltpu.SemaphoreType.DMA((N_DEV,)),
            pltpu.SemaphoreType.DMA((N_DEV,)),
            pltpu.SemaphoreType.DMA((N_DEV,)),
        ],
        compiler_params=pltpu.CompilerParams(collective_id=0),
    )(x2, wq, kt, vt, wo)
    return out.reshape(B, SQ_LOC, D_MODEL)


# device time: 13747 ns/iter; 1.9205x vs baseline; 1.9205x over previous
import os

import jax
import jax.numpy as jnp
from jax import lax
from jax.experimental import pallas as pl
from jax.experimental.pallas import tpu as pltpu

N_DEV = 4
B = 2
SQ_LOC = 128
D_MODEL = 512
HQ = 16
HQ_LOC = 4
DH = 64
SKV = 128
BLK = 64
GDIM = HQ_LOC * DH


def kernel(x, Wq, K_ext, V_ext, Wo):
    bf16 = jnp.bfloat16
    f32 = jnp.float32

    x2 = (x.reshape(B * SQ_LOC, D_MODEL) * 0.125).astype(bf16)
    kt = jnp.transpose(K_ext, (2, 0, 1, 3)).astype(bf16)
    vt = jnp.transpose(V_ext, (2, 0, 1, 3)).astype(bf16)
    wq = Wq.astype(bf16)
    wo = Wo.astype(bf16)

    def body(x_ref, wq_ref, kt_ref, vt_ref, wo_ref, out_ref,
             wq_slots, wo_slots, k_bd, v_bd,
             wq_send, wq_recv, wo_send, wo_recv):
        my = lax.axis_index("i")
        left = lax.rem(my + N_DEV - 1, N_DEV)
        right = lax.rem(my + 1, N_DEV)
        opp = lax.rem(my + 2, N_DEV)

        k_bd[...] = jnp.zeros((HQ_LOC * SKV, GDIM), bf16)
        v_bd[...] = jnp.zeros((HQ_LOC * SKV, GDIM), bf16)

        barrier_sem = pltpu.get_barrier_semaphore()
        for nbr in (left, right, opp):
            pl.semaphore_signal(
                barrier_sem, inc=1,
                device_id=(nbr,), device_id_type=pl.DeviceIdType.MESH,
            )
        pl.semaphore_wait(barrier_sem, N_DEV - 1)

        qi = lax.broadcasted_iota(jnp.int32, (SQ_LOC, HQ_LOC * SKV), 0)
        kj = lax.broadcasted_iota(jnp.int32, (SQ_LOC, HQ_LOC * SKV), 1)
        qb = my * (SQ_LOC // BLK) + qi // BLK
        kb = lax.rem(kj, SKV) // BLK
        mask = (qb == kb) | (kb == 0) | (lax.rem(qb + kb, 3) == 0)
        bias = jnp.where(mask, 0.0, -1e9).astype(f32)

        def compute_group(g, wq_g, wo_g, acc):
            q_g = jax.lax.dot_general(
                x_ref[...], wq_g, (((1,), (0,)), ((), ())),
                preferred_element_type=f32,
            ).astype(bf16)
            ctxs = []
            for b in range(B):
                for hh in range(HQ_LOC):
                    head = g * HQ_LOC + hh
                    kv_h = kt_ref[head]
                    vv_h = vt_ref[head]
                    k_bd[hh * SKV:(hh + 1) * SKV, hh * DH:(hh + 1) * DH] = kv_h[b]
                    v_bd[hh * SKV:(hh + 1) * SKV, hh * DH:(hh + 1) * DH] = vv_h[b]
                q_b = q_g[b * SQ_LOC:(b + 1) * SQ_LOC, :]
                s = jax.lax.dot_general(
                    q_b, k_bd[...], (((1,), (1,)), ((), ())),
                    preferred_element_type=f32,
                )
                w = jnp.exp(s + bias)
                w3 = w.reshape(SQ_LOC, HQ_LOC, SKV)
                w3 = w3 / jnp.sum(w3, axis=-1, keepdims=True)
                w2 = w3.reshape(SQ_LOC, HQ_LOC * SKV).astype(bf16)
                ctxs.append(jax.lax.dot_general(
                    w2, v_bd[...], (((1,), (0,)), ((), ())),
                    preferred_element_type=f32,
                ).astype(bf16))
            ctx = jnp.concatenate(ctxs, axis=0)
            return acc + jax.lax.dot_general(
                ctx, wo_g, (((1,), (0,)), ((), ())),
                preferred_element_type=f32,
            )

        acc = jnp.zeros((B * SQ_LOC, D_MODEL), dtype=f32)
        skip_comm = bool(os.environ.get("SKIP_COMM"))

        txs = []

        def push_to(dest):
            for src, slots, ssem, rsem in (
                (wq_ref, wq_slots, wq_send, wq_recv),
                (wo_ref, wo_slots, wo_send, wo_recv),
            ):
                tx = pltpu.make_async_remote_copy(
                    src_ref=src, dst_ref=slots.at[my],
                    send_sem=ssem.at[dest], recv_sem=rsem.at[my],
                    device_id=(dest,), device_id_type=pl.DeviceIdType.MESH,
                )
                tx.start()
                txs.append(tx)

        def wait_from(origin):
            for slots, rsem in ((wq_slots, wq_recv), (wo_slots, wo_recv)):
                rx = pltpu.make_async_remote_copy(
                    src_ref=slots.at[origin], dst_ref=slots.at[origin],
                    send_sem=wq_send.at[origin], recv_sem=rsem.at[origin],
                    device_id=(origin,), device_id_type=pl.DeviceIdType.MESH,
                )
                rx.wait_recv()

        if skip_comm:
            for g in range(N_DEV):
                acc = compute_group(g, wq_ref[...], wo_ref[...], acc)
        else:
            push_to(opp)
            push_to(right)
            push_to(left)
            acc = compute_group(my, wq_ref[...], wo_ref[...], acc)
            for origin in (left, right, opp):
                wait_from(origin)
                acc = compute_group(origin, wq_slots[origin],
                                    wo_slots[origin], acc)
            for tx in txs:
                tx.wait_send()

        out_ref[...] = acc

    out = pl.pallas_call(
        body,
        out_shape=jax.ShapeDtypeStruct((B * SQ_LOC, D_MODEL), jnp.float32),
        in_specs=[pl.BlockSpec(memory_space=pltpu.VMEM)] * 5,
        out_specs=pl.BlockSpec(memory_space=pltpu.VMEM),
        scratch_shapes=[
            pltpu.VMEM((N_DEV, D_MODEL, GDIM), bf16),
            pltpu.VMEM((N_DEV, GDIM, D_MODEL), bf16),
            pltpu.VMEM((HQ_LOC * SKV, GDIM), bf16),
            pltpu.VMEM((HQ_LOC * SKV, GDIM), bf16),
            pltpu.SemaphoreType.DMA((N_DEV,)),
            pltpu.SemaphoreType.DMA((N_DEV,)),
            pltpu.SemaphoreType.DMA((N_DEV,)),
            pltpu.SemaphoreType.DMA((N_DEV,)),
        ],
        compiler_params=pltpu.CompilerParams(collective_id=0),
    )(x2, wq, kt, vt, wo)
    return out.reshape(B, SQ_LOC, D_MODEL)
